# baseline (device time: 16115 ns/iter reference)
import math

import jax
import jax.numpy as jnp
from jax import lax
from jax.experimental import pallas as pl
from jax.experimental.pallas import tpu as pltpu

N_DEV = 4


def kernel(A, B):
    m, k_shard = A.shape
    _, n = B.shape
    m_out = m // N_DEV

    clip_t = 5.0 * math.sqrt(k_shard)
    q_scale = 127.0 / clip_t
    dq_scale = clip_t / 127.0

    def chunk_dot(a_ref, b_ref, c):
        return jnp.dot(
            a_ref[pl.ds(c * m_out, m_out), :],
            b_ref[:, :],
            preferred_element_type=jnp.float32,
        )

    def quantize(x):
        return jnp.round(
            jnp.clip(x * q_scale, -127.0, 127.0)
        ).astype(jnp.int8)

    def body(a_ref, b_ref, out_ref, s_ref, r_ref, send_sems, recv_sems):
        my = lax.axis_index("i")
        peers = [my ^ 1, 3 - my, my ^ 2]

        barrier_sem = pltpu.get_barrier_semaphore()
        for nbr in peers:
            pl.semaphore_signal(
                barrier_sem, inc=1,
                device_id=(nbr,), device_id_type=pl.DeviceIdType.MESH,
            )

        s_ref[0, :, :] = quantize(chunk_dot(a_ref, b_ref, peers[0]))
        pl.semaphore_wait(barrier_sem, 3)

        rdmas = []
        for k in range(3):
            if k > 0:
                s_ref[k, :, :] = quantize(chunk_dot(a_ref, b_ref, peers[k]))
            ch = pltpu.make_async_remote_copy(
                src_ref=s_ref.at[k], dst_ref=r_ref.at[k],
                send_sem=send_sems.at[k], recv_sem=recv_sems.at[k],
                device_id=(peers[k],), device_id_type=pl.DeviceIdType.MESH,
            )
            ch.start()
            rdmas.append(ch)

        own = chunk_dot(a_ref, b_ref, my)

        for r in rdmas:
            r.wait_recv()
        out_ref[:, :] = (
            own
            + (r_ref[0, :, :].astype(jnp.float32)
               + r_ref[1, :, :].astype(jnp.float32)
               + r_ref[2, :, :].astype(jnp.float32)) * dq_scale
        )

        for ch in rdmas:
            ch.wait_send()

    return pl.pallas_call(
        body,
        out_shape=jax.ShapeDtypeStruct((m_out, n), jnp.float32),
        in_specs=[
            pl.BlockSpec(memory_space=pltpu.VMEM),
            pl.BlockSpec(memory_space=pltpu.VMEM),
        ],
        out_specs=pl.BlockSpec(memory_space=pltpu.VMEM),
        scratch_shapes=[
            pltpu.VMEM((3, m_out, n), jnp.int8),
            pltpu.VMEM((3, m_out, n), jnp.int8),
            pltpu.SemaphoreType.DMA((3,)),
            pltpu.SemaphoreType.DMA((3,)),
        ],
        compiler_params=pltpu.CompilerParams(collective_id=0),
    )(A, B)


# device time: 15599 ns/iter; 1.0331x vs baseline; 1.0331x over previous
import math

import jax
import jax.numpy as jnp
from jax import lax
from jax.experimental import pallas as pl
from jax.experimental.pallas import tpu as pltpu

N_DEV = 4


def kernel(A, B):
    m, k_shard = A.shape
    _, n = B.shape
    m_out = m // N_DEV

    clip_t = 5.0 * math.sqrt(k_shard)
    q_scale = 127.0 / clip_t
    dq_scale = clip_t / 127.0

    def chunk_dot(a_ref, b_ref, c):
        return jnp.dot(
            a_ref[pl.ds(c * m_out, m_out), :],
            b_ref[:, :],
            preferred_element_type=jnp.float32,
        )

    def quantize(x):
        return jnp.round(
            jnp.clip(x * q_scale, -127.0, 127.0)
        ).astype(jnp.int8)

    def body(a_ref, b_ref, out_ref, s_ref, r_ref, send_sems, recv_sems):
        my = lax.axis_index("i")
        peers = [my ^ 1, 3 - my, my ^ 2]

        barrier_sem = pltpu.get_barrier_semaphore()
        for nbr in peers:
            pl.semaphore_signal(
                barrier_sem, inc=1,
                device_id=(nbr,), device_id_type=pl.DeviceIdType.MESH,
            )

        s_ref[0, :, :] = quantize(chunk_dot(a_ref, b_ref, peers[0]))
        pl.semaphore_wait(barrier_sem, 3)

        rdmas = []
        for k in range(3):
            if k > 0:
                s_ref[k, :, :] = quantize(chunk_dot(a_ref, b_ref, peers[k]))
            ch = pltpu.make_async_remote_copy(
                src_ref=s_ref.at[k], dst_ref=r_ref.at[k],
                send_sem=send_sems.at[k], recv_sem=recv_sems.at[k],
                device_id=(peers[k],), device_id_type=pl.DeviceIdType.MESH,
            )
            ch.start()
            rdmas.append(ch)

        own = chunk_dot(a_ref, b_ref, my)

        rdmas[0].wait_recv()
        rdmas[1].wait_recv()
        out_ref[:, :] = (
            own
            + (r_ref[0, :, :].astype(jnp.float32)
               + r_ref[1, :, :].astype(jnp.float32)) * dq_scale
        )

        rdmas[2].wait_recv()
        out_ref[:, :] = (
            out_ref[:, :] + r_ref[2, :, :].astype(jnp.float32) * dq_scale
        )

        for ch in rdmas:
            ch.wait_send()

    return pl.pallas_call(
        body,
        out_shape=jax.ShapeDtypeStruct((m_out, n), jnp.float32),
        in_specs=[
            pl.BlockSpec(memory_space=pltpu.VMEM),
            pl.BlockSpec(memory_space=pltpu.VMEM),
        ],
        out_specs=pl.BlockSpec(memory_space=pltpu.VMEM),
        scratch_shapes=[
            pltpu.VMEM((3, m_out, n), jnp.int8),
            pltpu.VMEM((3, m_out, n), jnp.int8),
            pltpu.SemaphoreType.DMA((3,)),
            pltpu.SemaphoreType.DMA((3,)),
        ],
        compiler_params=pltpu.CompilerParams(collective_id=0),
    )(A, B)
